# baseline (speedup 1.0000x reference)
import functools

import jax
import jax.numpy as jnp
from jax import lax
from jax.experimental import pallas as pl
from jax.experimental.pallas import tpu as pltpu
from jax.experimental.pallas import tpu_sc as plsc

N_SUB = 10000
E = 150000
D = 512
NCHUNK = 4
CW = D // NCHUNK
EPS = 1e-5

NC, NS = 2, 16
BLK = 128
NBLK = 76
EPT = NBLK * BLK
E_PAD = EPT * NS
STRIPE = 640


def _sc_aggregate_body(feat, idx4, dst, w, agg4, acc, *scratch):
    cid = lax.axis_index("c")
    sid = lax.axis_index("s")
    zero16 = jnp.zeros((16,), jnp.float32)
    idxs = scratch[0:4]
    dsts = scratch[4:8]
    ws = scratch[8:12]
    rows = scratch[12:14]
    rows0 = rows[0]
    lsems = scratch[14:18]
    gsems = scratch[18:20]
    ebase = sid * EPT

    def _load(c, g, b):
        off = ebase + g * BLK
        pltpu.async_copy(idx4.at[c, pl.ds(off, BLK)], idxs[b], lsems[b])
        pltpu.async_copy(dst.at[pl.ds(off, BLK)], dsts[b], lsems[b])
        pltpu.async_copy(w.at[pl.ds(off, BLK)], ws[b], lsems[b])

    def _wait_load(c, g, b):
        off = ebase + g * BLK
        pltpu.make_async_copy(idx4.at[c, pl.ds(off, BLK)], idxs[b], lsems[b]).wait()
        pltpu.make_async_copy(dst.at[pl.ds(off, BLK)], dsts[b], lsems[b]).wait()
        pltpu.make_async_copy(w.at[pl.ds(off, BLK)], ws[b], lsems[b]).wait()

    def _gather(lb, b):
        pltpu.async_copy(feat.at[idxs[lb]], rows[b], gsems[b])

    def _wait_gather(lb, b):
        pltpu.make_async_copy(feat.at[idxs[lb]], rows[b], gsems[b]).wait()

    def _mult(lb, b):
        def _egroup(g2, _):
            wv16 = ws[lb][pl.ds(16 * g2, 16)]
            for i in range(16):
                e = 16 * g2 + i
                wv = jnp.full((16,), wv16[i], jnp.float32)
                for j in range(CW // 16):
                    rows[b][e, pl.ds(16 * j, 16)] = rows[b][e, pl.ds(16 * j, 16)] * wv
            return 0

        lax.fori_loop(0, BLK // 16, _egroup, 0)

    def _scatter(lb, b):
        pltpu.sync_copy(rows[b], acc.at[dsts[lb]], add=True)

    for cc in range(NCHUNK // NC):
        c = cid * (NCHUNK // NC) + cc

        def _zrow(r, _):
            for j in range(CW // 16):
                rows0[r, pl.ds(16 * j, 16)] = zero16
            return 0

        lax.fori_loop(0, BLK, _zrow, 0)
        for k in range(STRIPE // 80):
            off = sid * STRIPE + k * 80

            @pl.when(off < N_SUB)
            def _():
                pltpu.sync_copy(rows0.at[pl.ds(0, 80)], acc.at[pl.ds(off, 80)])

        plsc.subcore_barrier()

        _load(c, 0, 0)
        _load(c, 1, 1)
        _load(c, 2, 2)
        _wait_load(c, 0, 0)
        _gather(0, 0)

        def _quad(t, _):
            gg = 4 * t
            for s in range(4):
                g = gg + s
                l1 = (s + 1) % 4
                l3 = (s + 3) % 4

                @pl.when(g + 1 < NBLK)
                def _():
                    _wait_load(c, g + 1, l1)
                    _gather(l1, (s + 1) % 2)

                _wait_gather(s, s % 2)
                _mult(s, s % 2)
                _scatter(s, s % 2)

                @pl.when(g + 3 < NBLK)
                def _():
                    _load(c, g + 3, l3)

            return 0

        lax.fori_loop(0, NBLK // 4, _quad, 0)
        plsc.subcore_barrier()

        for k in range(STRIPE // 80):
            off = sid * STRIPE + k * 80

            @pl.when(off < N_SUB)
            def _():
                pltpu.sync_copy(
                    acc.at[pl.ds(off, 80)],
                    agg4.at[c, pl.ds(off, 80)],
                )

        plsc.subcore_barrier()


@jax.jit
def _sc_aggregate(feat_flat, idx4, dst, w):
    mesh = plsc.VectorSubcoreMesh(core_axis_name="c", subcore_axis_name="s")
    f = pl.kernel(
        _sc_aggregate_body,
        out_type=jax.ShapeDtypeStruct((NCHUNK, N_SUB, CW), jnp.float32),
        mesh=mesh,
        scratch_types=(
            [pltpu.VMEM_SHARED((N_SUB, CW), jnp.float32)]
            + [pltpu.VMEM((BLK,), jnp.int32)] * 8
            + [pltpu.VMEM((BLK,), jnp.float32)] * 4
            + [pltpu.VMEM((BLK, CW), jnp.float32)] * 2
            + [pltpu.SemaphoreType.DMA] * 6
        ),
    )
    return f(feat_flat, idx4, dst, w)


ROWS_TC = 1000


def _mm_bn_body(agg_ref, wt_ref, b_ref, h_ref, st_ref):
    i = pl.program_id(0)
    x = jnp.concatenate([agg_ref[c] for c in range(NCHUNK)], axis=-1)
    h = jnp.dot(x, wt_ref[...], preferred_element_type=jnp.float32)
    h = jnp.maximum(h + b_ref[...], 0.0)
    h_ref[...] = h
    s = jnp.sum(h, axis=0, keepdims=True)
    sq = jnp.sum(h * h, axis=0, keepdims=True)
    st = jnp.concatenate([s, sq, jnp.zeros((6, D), jnp.float32)], axis=0)

    @pl.when(i == 0)
    def _():
        st_ref[...] = st

    @pl.when(i > 0)
    def _():
        st_ref[...] = st_ref[...] + st


@jax.jit
def _mm_bn(agg4, wt, b):
    grid = N_SUB // ROWS_TC
    return pl.pallas_call(
        _mm_bn_body,
        grid=(grid,),
        in_specs=[
            pl.BlockSpec((NCHUNK, ROWS_TC, CW), lambda i: (0, i, 0)),
            pl.BlockSpec((D, D), lambda i: (0, 0)),
            pl.BlockSpec((1, D), lambda i: (0, 0)),
        ],
        out_specs=[
            pl.BlockSpec((ROWS_TC, D), lambda i: (i, 0)),
            pl.BlockSpec((8, D), lambda i: (0, 0)),
        ],
        out_shape=[
            jax.ShapeDtypeStruct((N_SUB, D), jnp.float32),
            jax.ShapeDtypeStruct((8, D), jnp.float32),
        ],
    )(agg4, wt, b)


def _bn_body(h_ref, st_ref, g_ref, be_ref, o_ref):
    mean = st_ref[0:1, :] / N_SUB
    var = st_ref[1:2, :] / N_SUB - mean * mean
    a = g_ref[...] * lax.rsqrt(var + EPS)
    c = be_ref[...] - mean * a
    o_ref[...] = h_ref[...] * a + c


@jax.jit
def _bn_apply(h, st, g, be):
    grid = N_SUB // ROWS_TC
    return pl.pallas_call(
        _bn_body,
        grid=(grid,),
        in_specs=[
            pl.BlockSpec((ROWS_TC, D), lambda i: (i, 0)),
            pl.BlockSpec((8, D), lambda i: (0, 0)),
            pl.BlockSpec((1, D), lambda i: (0, 0)),
            pl.BlockSpec((1, D), lambda i: (0, 0)),
        ],
        out_specs=pl.BlockSpec((ROWS_TC, D), lambda i: (i, 0)),
        out_shape=jax.ShapeDtypeStruct((N_SUB, D), jnp.float32),
    )(h, st, g, be)


@jax.jit
def _bn_rows16(rows, st, g, be):
    return pl.pallas_call(
        _bn_body,
        grid=(1,),
        in_specs=[
            pl.BlockSpec((16, D), lambda i: (0, 0)),
            pl.BlockSpec((8, D), lambda i: (0, 0)),
            pl.BlockSpec((1, D), lambda i: (0, 0)),
            pl.BlockSpec((1, D), lambda i: (0, 0)),
        ],
        out_specs=pl.BlockSpec((16, D), lambda i: (0, 0)),
        out_shape=jax.ShapeDtypeStruct((16, D), jnp.float32),
    )(rows, st, g, be)


def kernel(edge_index, edge_weight, node_pair, node_features, W1, b1, g1, be1, W2, b2, g2, be2):
    B, NN, P, A, H = node_features.shape
    sub_feature = jnp.concatenate(
        [node_features[:, 0, 0, :, :][:, None, :, :], node_features[:, :, -1, :, :]],
        axis=1,
    ).reshape(-1, A * H)

    pad = E_PAD - E
    src = jnp.concatenate([edge_index[0], jnp.zeros((pad,), jnp.int32)])
    dst = jnp.concatenate([edge_index[1], jnp.zeros((pad,), jnp.int32)])
    w = jnp.concatenate([edge_weight[:, 0], jnp.zeros((pad,), jnp.float32)])
    idx4 = src[None, :] * NCHUNK + jnp.arange(NCHUNK, dtype=jnp.int32)[:, None]

    w1t = W1.T
    w2t = W2.T
    b1r = b1.reshape(1, D)
    b2r = b2.reshape(1, D)
    g1r = g1.reshape(1, D)
    be1r = be1.reshape(1, D)
    g2r = g2.reshape(1, D)
    be2r = be2.reshape(1, D)

    agg1 = _sc_aggregate(sub_feature.reshape(NCHUNK * N_SUB, CW), idx4, dst, w)
    h1, st1 = _mm_bn(agg1, w1t, b1r)
    h1n = _bn_apply(h1, st1, g1r, be1r)
    agg2 = _sc_aggregate(h1n.reshape(NCHUNK * N_SUB, CW), idx4, dst, w)
    h2, st2 = _mm_bn(agg2, w2t, b2r)
    rows16 = h2.reshape(B, NN + 1, D)[:, 0, :]
    out = _bn_rows16(rows16, st2, g2r, be2r)
    return out.reshape(B, A, H)


# speedup vs baseline: 1.5919x; 1.5919x over previous
import functools

import jax
import jax.numpy as jnp
from jax import lax
from jax.experimental import pallas as pl
from jax.experimental.pallas import tpu as pltpu
from jax.experimental.pallas import tpu_sc as plsc

N_SUB = 10000
E = 150000
D = 512
NCHUNK = 4
CW = D // NCHUNK
EPS = 1e-5

NC, NS = 2, 16
BLK = 128
NBLK = 74
EPT = NBLK * BLK
E_PAD = EPT * NS
STRIPE = 640


def _sc_aggregate_body(feat, idx4, dst, w, agg4, acc, *scratch):
    cid = lax.axis_index("c")
    sid = lax.axis_index("s")
    zero16 = jnp.zeros((16,), jnp.float32)
    idxs = scratch[0:2]
    dsts = scratch[2:4]
    ws = scratch[4:6]
    rows = scratch[6:8]
    rows0 = rows[0]
    lsems = scratch[8:10]
    gsems = scratch[10:12]
    ebase = sid * EPT

    def _load(c, g, b):
        off = ebase + g * BLK
        pltpu.async_copy(idx4.at[c, pl.ds(off, BLK)], idxs[b], lsems[b])
        pltpu.async_copy(dst.at[pl.ds(off, BLK)], dsts[b], lsems[b])
        pltpu.async_copy(w.at[pl.ds(off, BLK)], ws[b], lsems[b])

    def _wait_load(c, g, b):
        off = ebase + g * BLK
        pltpu.make_async_copy(idx4.at[c, pl.ds(off, BLK)], idxs[b], lsems[b]).wait()
        pltpu.make_async_copy(dst.at[pl.ds(off, BLK)], dsts[b], lsems[b]).wait()
        pltpu.make_async_copy(w.at[pl.ds(off, BLK)], ws[b], lsems[b]).wait()

    def _gather(lb, b):
        pltpu.async_copy(feat.at[idxs[lb]], rows[b], gsems[b])

    def _wait_gather(lb, b):
        pltpu.make_async_copy(feat.at[idxs[lb]], rows[b], gsems[b]).wait()

    def _mult(lb, b):
        def _egroup(g2, _):
            wv16 = ws[lb][pl.ds(16 * g2, 16)]
            for i in range(16):
                e = 16 * g2 + i
                wv = jnp.full((16,), wv16[i], jnp.float32)
                for j in range(CW // 16):
                    rows[b][e, pl.ds(16 * j, 16)] = rows[b][e, pl.ds(16 * j, 16)] * wv
            return 0

        lax.fori_loop(0, BLK // 16, _egroup, 0)

    def _scatter(lb, b):
        pltpu.sync_copy(rows[b], acc.at[dsts[lb]], add=True)

    for cc in range(NCHUNK // NC):
        c = cid * (NCHUNK // NC) + cc

        def _zrow(r, _):
            for j in range(CW // 16):
                rows0[r, pl.ds(16 * j, 16)] = zero16
            return 0

        lax.fori_loop(0, BLK, _zrow, 0)
        for k in range(STRIPE // 80):
            off = sid * STRIPE + k * 80

            @pl.when(off < N_SUB)
            def _():
                pltpu.sync_copy(rows0.at[pl.ds(0, 80)], acc.at[pl.ds(off, 80)])

        plsc.subcore_barrier()

        _load(c, 0, 0)
        _load(c, 1, 1)
        _wait_load(c, 0, 0)
        _gather(0, 0)

        def _pair(gp, _):
            g0 = 2 * gp
            _wait_load(c, g0 + 1, 1)
            _gather(1, 1)
            _wait_gather(0, 0)
            _mult(0, 0)
            _scatter(0, 0)
            _load(c, g0 + 2, 0)
            _wait_gather(1, 1)
            _mult(1, 1)
            _wait_load(c, g0 + 2, 0)
            _gather(0, 0)
            _scatter(1, 1)
            _load(c, g0 + 3, 1)
            return 0

        lax.fori_loop(0, NBLK // 2 - 1, _pair, 0)
        _wait_load(c, NBLK - 1, 1)
        _gather(1, 1)
        _wait_gather(0, 0)
        _mult(0, 0)
        _scatter(0, 0)
        _wait_gather(1, 1)
        _mult(1, 1)
        _scatter(1, 1)
        plsc.subcore_barrier()

        for k in range(STRIPE // 80):
            off = sid * STRIPE + k * 80

            @pl.when(off < N_SUB)
            def _():
                pltpu.sync_copy(
                    acc.at[pl.ds(off, 80)],
                    agg4.at[c, pl.ds(off, 80)],
                )

        plsc.subcore_barrier()


@jax.jit
def _sc_aggregate(feat_flat, idx4, dst, w):
    mesh = plsc.VectorSubcoreMesh(core_axis_name="c", subcore_axis_name="s")
    f = pl.kernel(
        _sc_aggregate_body,
        out_type=jax.ShapeDtypeStruct((NCHUNK, N_SUB, CW), jnp.float32),
        mesh=mesh,
        scratch_types=(
            [pltpu.VMEM_SHARED((N_SUB, CW), jnp.float32)]
            + [pltpu.VMEM((BLK,), jnp.int32)] * 4
            + [pltpu.VMEM((BLK,), jnp.float32)] * 2
            + [pltpu.VMEM((BLK, CW), jnp.float32)] * 2
            + [pltpu.SemaphoreType.DMA] * 4
        ),
    )
    return f(feat_flat, idx4, dst, w)


ROWS_TC = 1000


def _mm_bn_body(agg_ref, wt_ref, b_ref, h_ref, st_ref):
    i = pl.program_id(0)
    x = jnp.concatenate([agg_ref[c] for c in range(NCHUNK)], axis=-1)
    h = jnp.dot(x, wt_ref[...], preferred_element_type=jnp.float32)
    h = jnp.maximum(h + b_ref[...], 0.0)
    h_ref[...] = h
    s = jnp.sum(h, axis=0, keepdims=True)
    sq = jnp.sum(h * h, axis=0, keepdims=True)
    st = jnp.concatenate([s, sq, jnp.zeros((6, D), jnp.float32)], axis=0)

    @pl.when(i == 0)
    def _():
        st_ref[...] = st

    @pl.when(i > 0)
    def _():
        st_ref[...] = st_ref[...] + st


@jax.jit
def _mm_bn(agg4, wt, b):
    grid = N_SUB // ROWS_TC
    return pl.pallas_call(
        _mm_bn_body,
        grid=(grid,),
        in_specs=[
            pl.BlockSpec((NCHUNK, ROWS_TC, CW), lambda i: (0, i, 0)),
            pl.BlockSpec((D, D), lambda i: (0, 0)),
            pl.BlockSpec((1, D), lambda i: (0, 0)),
        ],
        out_specs=[
            pl.BlockSpec((ROWS_TC, D), lambda i: (i, 0)),
            pl.BlockSpec((8, D), lambda i: (0, 0)),
        ],
        out_shape=[
            jax.ShapeDtypeStruct((N_SUB, D), jnp.float32),
            jax.ShapeDtypeStruct((8, D), jnp.float32),
        ],
    )(agg4, wt, b)


def _bn_body(h_ref, st_ref, g_ref, be_ref, o_ref):
    mean = st_ref[0:1, :] / N_SUB
    var = st_ref[1:2, :] / N_SUB - mean * mean
    a = g_ref[...] * lax.rsqrt(var + EPS)
    c = be_ref[...] - mean * a
    o_ref[...] = h_ref[...] * a + c


@jax.jit
def _bn_apply(h, st, g, be):
    grid = N_SUB // ROWS_TC
    return pl.pallas_call(
        _bn_body,
        grid=(grid,),
        in_specs=[
            pl.BlockSpec((ROWS_TC, D), lambda i: (i, 0)),
            pl.BlockSpec((8, D), lambda i: (0, 0)),
            pl.BlockSpec((1, D), lambda i: (0, 0)),
            pl.BlockSpec((1, D), lambda i: (0, 0)),
        ],
        out_specs=pl.BlockSpec((ROWS_TC, D), lambda i: (i, 0)),
        out_shape=jax.ShapeDtypeStruct((N_SUB, D), jnp.float32),
    )(h, st, g, be)


@jax.jit
def _bn_rows16(rows, st, g, be):
    return pl.pallas_call(
        _bn_body,
        grid=(1,),
        in_specs=[
            pl.BlockSpec((16, D), lambda i: (0, 0)),
            pl.BlockSpec((8, D), lambda i: (0, 0)),
            pl.BlockSpec((1, D), lambda i: (0, 0)),
            pl.BlockSpec((1, D), lambda i: (0, 0)),
        ],
        out_specs=pl.BlockSpec((16, D), lambda i: (0, 0)),
        out_shape=jax.ShapeDtypeStruct((16, D), jnp.float32),
    )(rows, st, g, be)


def kernel(edge_index, edge_weight, node_pair, node_features, W1, b1, g1, be1, W2, b2, g2, be2):
    B, NN, P, A, H = node_features.shape
    sub_feature = jnp.concatenate(
        [node_features[:, 0, 0, :, :][:, None, :, :], node_features[:, :, -1, :, :]],
        axis=1,
    ).reshape(-1, A * H)

    pad = E_PAD - E
    src = jnp.concatenate([edge_index[0], jnp.zeros((pad,), jnp.int32)])
    dst = jnp.concatenate([edge_index[1], jnp.zeros((pad,), jnp.int32)])
    w = jnp.concatenate([edge_weight[:, 0], jnp.zeros((pad,), jnp.float32)])
    idx4 = src[None, :] * NCHUNK + jnp.arange(NCHUNK, dtype=jnp.int32)[:, None]

    w1t = W1.T
    w2t = W2.T
    b1r = b1.reshape(1, D)
    b2r = b2.reshape(1, D)
    g1r = g1.reshape(1, D)
    be1r = be1.reshape(1, D)
    g2r = g2.reshape(1, D)
    be2r = be2.reshape(1, D)

    agg1 = _sc_aggregate(sub_feature.reshape(NCHUNK * N_SUB, CW), idx4, dst, w)
    h1, st1 = _mm_bn(agg1, w1t, b1r)
    h1n = _bn_apply(h1, st1, g1r, be1r)
    agg2 = _sc_aggregate(h1n.reshape(NCHUNK * N_SUB, CW), idx4, dst, w)
    h2, st2 = _mm_bn(agg2, w2t, b2r)
    rows16 = h2.reshape(B, NN + 1, D)[:, 0, :]
    out = _bn_rows16(rows16, st2, g2r, be2r)
    return out.reshape(B, A, H)
